# baseline (speedup 1.0000x reference)
import functools
import math

import jax
import jax.numpy as jnp
from jax import lax
from jax.experimental import pallas as pl
from jax.experimental.pallas import tpu as pltpu
from jax.experimental.pallas import tpu_sc as plsc

D = 128
L = 16
NC = 2
NS = 16
NW = NC * NS
H0 = 128
NBUF = 8
SCALE = float(math.sqrt(float(D)))


@jax.jit
def _embed(x, table):
    R, C = x.shape
    H1 = C - H0
    rows_per_w = R // NW
    n_chunks = 2 * rows_per_w
    assert n_chunks % NBUF == 0 and NBUF % 2 == 0

    mesh = plsc.VectorSubcoreMesh(core_axis_name="c", subcore_axis_name="s")

    scratch = [
        pltpu.VMEM((rows_per_w, C), jnp.int32),
    ]
    scratch += [
        pltpu.VMEM((H0 if b % 2 == 0 else H1, D), jnp.float32)
        for b in range(NBUF)
    ]
    scratch += [pltpu.SemaphoreType.DMA for _ in range(2 * NBUF)]

    @functools.partial(
        pl.kernel,
        mesh=mesh,
        out_type=jax.ShapeDtypeStruct((R, C, D), jnp.float32),
        scratch_types=scratch,
    )
    def k(x_hbm, table_hbm, out_hbm, idx_v, *bufs_and_sems):
        rows = bufs_and_sems[:NBUF]
        gsem = bufs_and_sems[NBUF:2 * NBUF]
        ssem = bufs_and_sems[2 * NBUF:3 * NBUF]

        wid = lax.axis_index("s") * NC + lax.axis_index("c")
        xrow0 = wid * rows_per_w

        pltpu.sync_copy(x_hbm.at[pl.ds(xrow0, rows_per_w)], idx_v)

        def halves(b):
            lo = 0 if b % 2 == 0 else H0
            n = H0 if b % 2 == 0 else H1
            return lo, n

        def gather(g, b):
            lo, n = halves(b)
            return pltpu.make_async_copy(
                table_hbm.at[idx_v.at[g // 2, pl.ds(lo, n)]],
                rows[b].at[pl.ds(0, n)],
                gsem[b])

        def store(g, b):
            lo, n = halves(b)
            return pltpu.make_async_copy(
                rows[b].at[pl.ds(0, n)],
                out_hbm.at[xrow0 + g // 2, pl.ds(lo, n)],
                ssem[b])

        for b in range(NBUF - 2):
            gather(b, b).start()

        def round_body(go, carry):
            for b in range(NBUF):
                g = go * NBUF + b
                bp = (b - 2) % NBUF

                @pl.when(g + NBUF - 2 < n_chunks)
                def _():
                    @pl.when(g >= 2)
                    def _():
                        store(lax.max(g - 2, 0), bp).wait()
                    gather(g + NBUF - 2, bp).start()

                gather(g, b).wait()

                _, n = halves(b)

                def scale_oct(i, c2):
                    for r in range(8):
                        for v in range(D // L):
                            rows[b][i * 8 + r, pl.ds(v * L, L)] = (
                                rows[b][i * 8 + r, pl.ds(v * L, L)] * SCALE
                            )
                    return c2

                lax.fori_loop(0, n // 8, scale_oct, 0)
                store(g, b).start()
            return carry

        lax.fori_loop(0, n_chunks // NBUF, round_body, 0)

        for b in range(NBUF):
            store(n_chunks - NBUF + b, b).wait()

    return k(x, table)


def kernel(x, table):
    return _embed(x, table)


# speedup vs baseline: 1.0076x; 1.0076x over previous
import functools
import math

import jax
import jax.numpy as jnp
from jax import lax
from jax.experimental import pallas as pl
from jax.experimental.pallas import tpu as pltpu
from jax.experimental.pallas import tpu_sc as plsc

D = 128
L = 16
NC = 2
NS = 16
NW = NC * NS
H0 = 128
NBUF = 8
SCALE = float(math.sqrt(float(D)))


@jax.jit
def _embed(x, table):
    R, C = x.shape
    H1 = C - H0
    rows_per_w = R // NW
    n_chunks = 2 * rows_per_w
    assert n_chunks % NBUF == 0 and NBUF % 2 == 0

    mesh = plsc.VectorSubcoreMesh(core_axis_name="c", subcore_axis_name="s")

    scratch = [
        pltpu.VMEM((rows_per_w, C), jnp.int32),
    ]
    scratch += [
        pltpu.VMEM((H0 if b % 2 == 0 else H1, D), jnp.float32)
        for b in range(NBUF)
    ]
    scratch += [pltpu.SemaphoreType.DMA for _ in range(2 * NBUF)]

    @functools.partial(
        pl.kernel,
        mesh=mesh,
        out_type=jax.ShapeDtypeStruct((R, C, D), jnp.float32),
        scratch_types=scratch,
        compiler_params=pltpu.CompilerParams(use_tc_tiling_on_sc=True),
    )
    def k(x_hbm, table_hbm, out_hbm, idx_v, *bufs_and_sems):
        rows = bufs_and_sems[:NBUF]
        gsem = bufs_and_sems[NBUF:2 * NBUF]
        ssem = bufs_and_sems[2 * NBUF:3 * NBUF]

        wid = lax.axis_index("s") * NC + lax.axis_index("c")
        xrow0 = wid * rows_per_w

        pltpu.sync_copy(x_hbm.at[pl.ds(xrow0, rows_per_w)], idx_v)

        def halves(b):
            lo = 0 if b % 2 == 0 else H0
            n = H0 if b % 2 == 0 else H1
            return lo, n

        def gather(g, b):
            lo, n = halves(b)
            return pltpu.make_async_copy(
                table_hbm.at[idx_v.at[g // 2, pl.ds(lo, n)]],
                rows[b].at[pl.ds(0, n)],
                gsem[b])

        def store(g, b):
            lo, n = halves(b)
            return pltpu.make_async_copy(
                rows[b].at[pl.ds(0, n)],
                out_hbm.at[xrow0 + g // 2, pl.ds(lo, n)],
                ssem[b])

        for b in range(NBUF - 2):
            gather(b, b).start()

        def round_body(go, carry):
            for b in range(NBUF):
                g = go * NBUF + b
                bp = (b - 2) % NBUF

                @pl.when(g + NBUF - 2 < n_chunks)
                def _():
                    @pl.when(g >= 2)
                    def _():
                        store(lax.max(g - 2, 0), bp).wait()
                    gather(g + NBUF - 2, bp).start()

                gather(g, b).wait()

                _, n = halves(b)

                def scale_quad(i, c2):
                    for r in range(4):
                        for v in range(D // L):
                            rows[b][i * 4 + r, pl.ds(v * L, L)] = (
                                rows[b][i * 4 + r, pl.ds(v * L, L)] * SCALE
                            )
                    return c2

                lax.fori_loop(0, n // 4, scale_quad, 0)
                store(g, b).start()
            return carry

        lax.fori_loop(0, n_chunks // NBUF, round_body, 0)

        for b in range(NBUF):
            store(n_chunks - NBUF + b, b).wait()

    return k(x, table)


def kernel(x, table):
    return _embed(x, table)


# speedup vs baseline: 1.0081x; 1.0005x over previous
import functools
import math

import jax
import jax.numpy as jnp
from jax import lax
from jax.experimental import pallas as pl
from jax.experimental.pallas import tpu as pltpu
from jax.experimental.pallas import tpu_sc as plsc

D = 128
L = 16
NC = 2
NS = 16
NW = NC * NS
H0 = 128
NBUF = 8
SCALE = float(math.sqrt(float(D)))


@jax.jit
def _embed(x, table):
    R, C = x.shape
    H1 = C - H0
    rows_per_w = R // NW
    n_chunks = 2 * rows_per_w
    assert n_chunks % NBUF == 0 and NBUF % 2 == 0

    mesh = plsc.VectorSubcoreMesh(core_axis_name="c", subcore_axis_name="s")

    scratch = [
        pltpu.VMEM((rows_per_w, C), jnp.int32),
    ]
    scratch += [
        pltpu.VMEM((H0 if b % 2 == 0 else H1, D), jnp.float32)
        for b in range(NBUF)
    ]
    scratch += [pltpu.SemaphoreType.DMA for _ in range(2 * NBUF)]

    @functools.partial(
        pl.kernel,
        mesh=mesh,
        out_type=jax.ShapeDtypeStruct((R, C, D), jnp.float32),
        scratch_types=scratch,
    )
    def k(x_hbm, table_hbm, out_hbm, idx_v, *bufs_and_sems):
        rows = bufs_and_sems[:NBUF]
        gsem = bufs_and_sems[NBUF:2 * NBUF]
        ssem = bufs_and_sems[2 * NBUF:3 * NBUF]

        wid = lax.axis_index("s") * NC + lax.axis_index("c")
        xrow0 = wid * rows_per_w

        pltpu.sync_copy(x_hbm.at[pl.ds(xrow0, rows_per_w)], idx_v)

        def halves(b):
            lo = 0 if b % 2 == 0 else H0
            n = H0 if b % 2 == 0 else H1
            return lo, n

        def gather(g, b):
            lo, n = halves(b)
            return pltpu.make_async_copy(
                table_hbm.at[idx_v.at[g // 2, pl.ds(lo, n)]],
                rows[b].at[pl.ds(0, n)],
                gsem[b])

        def store(g, b):
            lo, n = halves(b)
            return pltpu.make_async_copy(
                rows[b].at[pl.ds(0, n)],
                out_hbm.at[xrow0 + g // 2, pl.ds(lo, n)],
                ssem[b])

        for b in range(NBUF - 2):
            gather(b, b).start()

        def round_body(go, carry):
            for b in range(NBUF):
                g = go * NBUF + b
                bp = (b - 2) % NBUF

                @pl.when(g + NBUF - 2 < n_chunks)
                def _():
                    @pl.when(g >= 2)
                    def _():
                        store(lax.max(g - 2, 0), bp).wait()
                    gather(g + NBUF - 2, bp).start()

                gather(g, b).wait()

                _, n = halves(b)

                def scale_quad(i, c2):
                    for r in range(4):
                        for v in range(D // L):
                            rows[b][i * 4 + r, pl.ds(v * L, L)] = (
                                rows[b][i * 4 + r, pl.ds(v * L, L)] * SCALE
                            )
                    return c2

                lax.fori_loop(0, n // 4, scale_quad, 0)
                store(g, b).start()
            return carry

        lax.fori_loop(0, n_chunks // NBUF, round_body, 0)

        for b in range(NBUF):
            store(n_chunks - NBUF + b, b).wait()

    return k(x, table)


def kernel(x, table):
    return _embed(x, table)
